# baseline (speedup 1.0000x reference)
import functools
import jax
import jax.numpy as jnp
from jax.experimental import pallas as pl
from jax.experimental.pallas import tpu as pltpu

HID = 128
HALF = 64
BN = 400
EDGE_CHUNK = 25000


def _lane_iota():
    return jax.lax.broadcasted_iota(jnp.int32, (1, HID), 1)


def _lin_kernel(x_ref, w_ref, b_ref, o_ref):
    y = jnp.dot(x_ref[...], w_ref[...], preferred_element_type=jnp.float32)
    o_ref[...] = jnp.maximum(y + b_ref[...], 0.0)


def _lin(x, w, b):
    n = x.shape[0]
    return pl.pallas_call(
        _lin_kernel,
        grid=(n // BN,),
        in_specs=[
            pl.BlockSpec((BN, x.shape[1]), lambda i: (i, 0)),
            pl.BlockSpec((x.shape[1], HID), lambda i: (0, 0)),
            pl.BlockSpec((1, HID), lambda i: (0, 0)),
        ],
        out_specs=pl.BlockSpec((BN, HID), lambda i: (i, 0)),
        out_shape=jax.ShapeDtypeStruct((n, HID), jnp.float32),
    )(x, w, b.reshape(1, HID))


def _mm_kernel(x_ref, w_ref, o_ref):
    o_ref[...] = jnp.dot(x_ref[...], w_ref[...],
                         preferred_element_type=jnp.float32)


def _mm(x, w):
    n = x.shape[0]
    return pl.pallas_call(
        _mm_kernel,
        grid=(n // BN,),
        in_specs=[
            pl.BlockSpec((BN, HID), lambda i: (i, 0)),
            pl.BlockSpec((HID, HID), lambda i: (0, 0)),
        ],
        out_specs=pl.BlockSpec((BN, HID), lambda i: (i, 0)),
        out_shape=jax.ShapeDtypeStruct((n, HID), jnp.float32),
    )(x, w)


def _adst_kernel(x_ref, w_ref, att_ref, o_ref):
    hd = jnp.dot(x_ref[...], w_ref[...], preferred_element_type=jnp.float32)
    prod = hd * att_ref[...]
    o_ref[:, 0:1] = jnp.sum(prod[:, :HALF], axis=1, keepdims=True)
    o_ref[:, 1:2] = jnp.sum(prod[:, HALF:], axis=1, keepdims=True)


def _adst(x, w, att_flat):
    n = x.shape[0]
    return pl.pallas_call(
        _adst_kernel,
        grid=(n // BN,),
        in_specs=[
            pl.BlockSpec((BN, HID), lambda i: (i, 0)),
            pl.BlockSpec((HID, HID), lambda i: (0, 0)),
            pl.BlockSpec((1, HID), lambda i: (0, 0)),
        ],
        out_specs=pl.BlockSpec((BN, 2), lambda i: (i, 0)),
        out_shape=jax.ShapeDtypeStruct((n, 2), jnp.float32),
    )(x, w, att_flat)


def _edge_kernel(src_ref, dst_ref, hs_ref, adp_ref, att_ref,
                 out_ref, den_ref, *, n_src):
    lanes = _lane_iota()
    mask_lo = lanes < HALF

    out_ref[...] = jnp.zeros_like(out_ref)
    den_ref[...] = jnp.zeros_like(den_ref)

    att = att_ref[...]
    CH = 500
    n_it = n_src // CH

    def mx_body(i, acc):
        blk = hs_ref[pl.ds(i * CH, CH), :] * att
        return jnp.maximum(acc, jnp.max(blk, axis=0, keepdims=True))

    colmax = jax.lax.fori_loop(
        0, n_it, mx_body, jnp.full((1, HID), -1e30, jnp.float32))
    ms0 = jnp.sum(jnp.where(mask_lo, colmax, 0.0))
    ms1 = jnp.sum(jnp.where(mask_lo, 0.0, colmax))

    npk = adp_ref.shape[0]
    PCH = 16

    def admx_body(i, acc):
        return jnp.maximum(
            acc, jnp.max(adp_ref[pl.ds(i * PCH, PCH), :], axis=0,
                         keepdims=True))

    admax = jax.lax.fori_loop(
        0, npk // PCH, admx_body, jnp.full((1, HID), -1e30, jnp.float32))
    even = (lanes % 2) == 0
    ma0 = jnp.max(jnp.where(even, admax, -1e30))
    ma1 = jnp.max(jnp.where(even, -1e30, admax))
    m0 = jnp.maximum(ms0 + ma0, 0.0)
    m1 = jnp.maximum(ms1 + ma1, 0.0)
    m_vec = jnp.where(mask_lo, m0, m1)

    n_edges = src_ref.shape[0]

    def body(e, carry):
        s = src_ref[e]
        d = dst_ref[e]
        hrow = hs_ref[pl.ds(s, 1), :]
        prod = hrow * att
        as0 = jnp.sum(jnp.where(mask_lo, prod, 0.0))
        as1 = jnp.sum(jnp.where(mask_lo, 0.0, prod))
        dr = d // HALF
        dm = d - dr * HALF
        adrow = adp_ref[pl.ds(dr, 1), :]
        sel0 = lanes == (2 * dm)
        sel1 = lanes == (2 * dm + 1)
        ad0 = jnp.sum(jnp.where(sel0, adrow, 0.0))
        ad1 = jnp.sum(jnp.where(sel1, adrow, 0.0))
        a_vec = jnp.where(mask_lo, as0 + ad0, as1 + ad1)
        a_vec = jnp.where(a_vec > 0, a_vec, 0.2 * a_vec)
        ex_vec = jnp.exp(a_vec - m_vec)
        out_ref[pl.ds(d, 1), :] += hrow * ex_vec
        e0 = jnp.max(jnp.where(mask_lo, ex_vec, 0.0))
        e1 = jnp.max(jnp.where(mask_lo, 0.0, ex_vec))
        den_ref[pl.ds(dr, 1), :] += (jnp.where(sel0, e0, 0.0)
                                     + jnp.where(sel1, e1, 0.0))
        return carry

    jax.lax.fori_loop(0, n_edges, body, 0)


def _gat_edges(hs, adst, att_src_flat, src, dst, n_dst):
    n_src = hs.shape[0]
    n_pad = ((n_dst + 511) // 512) * 512
    adp = jnp.zeros((n_pad, 2), jnp.float32).at[:n_dst].set(adst)
    adp = adp.reshape(n_pad // HALF, HID)
    npk = adp.shape[0]

    kern = functools.partial(_edge_kernel, n_src=n_src)
    out, den = pl.pallas_call(
        kern,
        grid_spec=pltpu.PrefetchScalarGridSpec(
            num_scalar_prefetch=2,
            grid=(1,),
            in_specs=[
                pl.BlockSpec((n_src, HID), lambda i, s, d: (0, 0)),
                pl.BlockSpec((npk, HID), lambda i, s, d: (0, 0)),
                pl.BlockSpec((1, HID), lambda i, s, d: (0, 0)),
            ],
            out_specs=[
                pl.BlockSpec((n_dst, HID), lambda i, s, d: (0, 0)),
                pl.BlockSpec((npk, HID), lambda i, s, d: (0, 0)),
            ],
        ),
        out_shape=[
            jax.ShapeDtypeStruct((n_dst, HID), jnp.float32),
            jax.ShapeDtypeStruct((npk, HID), jnp.float32),
        ],
    )(src, dst, hs, adp, att_src_flat)

    den2 = den.reshape(-1, 2)[:n_dst]
    return out, den2


def _epi_kernel(res_ref, o1_ref, d1_ref, b1_ref, o2_ref, d2_ref, b2_ref,
                g_ref, bt_ref, out_ref):
    lanes = _lane_iota()
    mask_lo = lanes < HALF

    def norm_msg(o_ref, d_ref, b_ref):
        d = d_ref[...]
        dexp = jnp.where(mask_lo, d[:, 0:1], d[:, 1:2])
        return o_ref[...] / (dexp + 1e-16) + b_ref[...]

    y = norm_msg(o1_ref, d1_ref, b1_ref) + norm_msg(o2_ref, d2_ref, b2_ref)
    y = y + res_ref[...]
    mu = jnp.mean(y, axis=1, keepdims=True)
    var = jnp.mean((y - mu) ** 2, axis=1, keepdims=True)
    y = (y - mu) * jax.lax.rsqrt(var + 1e-5) * g_ref[...] + bt_ref[...]
    out_ref[...] = jnp.maximum(y, 0.0)


def _epilogue(res, o1, d1, b1, o2, d2, b2, gamma, beta):
    n = res.shape[0]
    full = lambda i: (i, 0)
    zero = lambda i: (0, 0)
    return pl.pallas_call(
        _epi_kernel,
        grid=(n // BN,),
        in_specs=[
            pl.BlockSpec((BN, HID), full),
            pl.BlockSpec((BN, HID), full),
            pl.BlockSpec((BN, 2), full),
            pl.BlockSpec((1, HID), zero),
            pl.BlockSpec((BN, HID), full),
            pl.BlockSpec((BN, 2), full),
            pl.BlockSpec((1, HID), zero),
            pl.BlockSpec((1, HID), zero),
            pl.BlockSpec((1, HID), zero),
        ],
        out_specs=pl.BlockSpec((BN, HID), full),
        out_shape=jax.ShapeDtypeStruct((n, HID), jnp.float32),
    )(res, o1, d1, b1.reshape(1, HID), o2, d2, b2.reshape(1, HID),
      gamma.reshape(1, HID), beta.reshape(1, HID))


def _gat(h_src, h_dst, ei, p):
    n_dst = h_dst.shape[0]
    hs = _mm(h_src, p['W_src'])
    adst = _adst(h_dst, p['W_dst'], p['att_dst'].reshape(1, HID))
    out, den = _gat_edges(hs, adst, p['att_src'].reshape(1, HID),
                          ei[0], ei[1], n_dst)
    return out, den


def kernel(x_idle, x_quasi, x_task, ei_idle_idle, ei_idle_quasi,
           ei_quasi_idle, ei_quasi_task, ei_task_quasi, params):
    h = {
        'idle': _lin(x_idle, params['lin']['idle']['W'],
                     params['lin']['idle']['b']),
        'quasi': _lin(x_quasi, params['lin']['quasi']['W'],
                      params['lin']['quasi']['b']),
        'task': _lin(x_task, params['lin']['task']['W'],
                     params['lin']['task']['b']),
    }
    for layer in params['layers']:
        c = layer['conv']
        o_ii, d_ii = _gat(h['idle'], h['idle'], ei_idle_idle, c['idle__idle'])
        o_qi, d_qi = _gat(h['quasi'], h['idle'], ei_quasi_idle,
                          c['quasi__idle'])
        o_iq, d_iq = _gat(h['idle'], h['quasi'], ei_idle_quasi,
                          c['idle__quasi'])
        o_tq, d_tq = _gat(h['task'], h['quasi'], ei_task_quasi,
                          c['task__quasi'])
        o_qt, d_qt = _gat(h['quasi'], h['task'], ei_quasi_task,
                          c['quasi__task'])
        nrm = layer['norm']
        n_task = h['task'].shape[0]
        zo = jnp.zeros((n_task, HID), jnp.float32)
        zd = jnp.ones((n_task, 2), jnp.float32)
        zb = jnp.zeros((HID,), jnp.float32)
        new_idle = _epilogue(h['idle'], o_ii, d_ii, c['idle__idle']['bias'],
                             o_qi, d_qi, c['quasi__idle']['bias'],
                             nrm['idle']['gamma'], nrm['idle']['beta'])
        new_quasi = _epilogue(h['quasi'], o_iq, d_iq, c['idle__quasi']['bias'],
                              o_tq, d_tq, c['task__quasi']['bias'],
                              nrm['quasi']['gamma'], nrm['quasi']['beta'])
        new_task = _epilogue(h['task'], o_qt, d_qt, c['quasi__task']['bias'],
                             zo, zd, zb,
                             nrm['task']['gamma'], nrm['task']['beta'])
        h['idle'], h['quasi'], h['task'] = new_idle, new_quasi, new_task
    return (h['idle'], h['quasi'], h['task'])
